# baseline (speedup 1.0000x reference)
import functools

import jax
import jax.numpy as jnp
from jax import lax
from jax.experimental import pallas as pl
from jax.experimental.pallas import tpu as pltpu
from jax.experimental.pallas import tpu_sc as plsc

_N = 2097152
_K = 209715
_NMK = _N - _K
_ROWS = 2048
_COLS = 1024
_CHUNKS = 8
_CROWS = _ROWS // _CHUNKS

_NW = 32
_PER_W = _N // _NW
_SC_CHUNK = 16384
_BINS = 34816
_BIN_ROWS = 272

_REFINE_ITERS = 7
_SLICES = 16
_SROWS = _ROWS // _SLICES


def _tc1_body(p_ref, t_ref, bits_ref, dice_ref, acc_ref):
    i = pl.program_id(0)

    @pl.when(i == 0)
    def _init():
        acc_ref[0] = 0.0
        acc_ref[1] = 0.0
        acc_ref[2] = 0.0

    sp, st, si = [], [], []
    for j in range(4):
        rows = pl.ds(j * (_CROWS // 4), _CROWS // 4)
        p = p_ref[rows, :]
        t = t_ref[rows, :]
        pt = p * t
        sp.append(jnp.sum(p))
        st.append(jnp.sum(t))
        si.append(jnp.sum(pt))
        q = (1.0 - p) - t + 2.0 * pt
        bce = jnp.maximum(jnp.minimum(-jnp.log(q), 100.0), 0.0)
        bits_ref[rows, :] = pltpu.bitcast(bce, jnp.int32)
    acc_ref[0] += sum(sp)
    acc_ref[1] += sum(st)
    acc_ref[2] += sum(si)

    @pl.when(i == _CHUNKS - 1)
    def _fin():
        dice = 1.0 - (2.0 * acc_ref[2] + 1.0) / (acc_ref[0] + acc_ref[1] + 1.0)
        dice_ref[...] = dice.reshape(1, 1)


def _tc1(p, t):
    return pl.pallas_call(
        _tc1_body,
        grid=(_CHUNKS,),
        in_specs=[
            pl.BlockSpec((_CROWS, _COLS), lambda i: (i, 0)),
            pl.BlockSpec((_CROWS, _COLS), lambda i: (i, 0)),
        ],
        out_specs=[
            pl.BlockSpec((_CROWS, _COLS), lambda i: (i, 0)),
            pl.BlockSpec((1, 1), lambda i: (0, 0)),
        ],
        out_shape=[
            jax.ShapeDtypeStruct((_ROWS, _COLS), jnp.int32),
            jax.ShapeDtypeStruct((1, 1), jnp.float32),
        ],
        scratch_shapes=[pltpu.SMEM((4,), jnp.float32)],
    )(p, t)


def _sc_hist_body(bits_hbm, out_hbm, chunk_v, hist_v):
    wid = lax.axis_index("s") * 2 + lax.axis_index("c")
    base = wid * _PER_W

    def zero_step(z, _):
        hist_v[pl.ds(z * 16, 16)] = jnp.zeros((16,), jnp.int32)
        return ()

    lax.fori_loop(0, _BINS // 16, zero_step, ())

    ones = jnp.ones((16,), jnp.int32)
    for c in range(_PER_W // _SC_CHUNK):
        pltpu.sync_copy(
            bits_hbm.at[pl.ds(base + c * _SC_CHUNK, _SC_CHUNK)], chunk_v
        )

        def step(j, _):
            for u in range(4):
                v = chunk_v[pl.ds((j * 4 + u) * 16, 16)]
                bins = lax.shift_right_logical(v, 15)
                plsc.addupdate_scatter(hist_v, [bins], ones)
            return ()

        lax.fori_loop(0, _SC_CHUNK // 64, step, ())

    pltpu.sync_copy(hist_v, out_hbm.at[wid])


def _sc_hist(bits_flat):
    mesh = plsc.VectorSubcoreMesh(core_axis_name="c", subcore_axis_name="s")
    kfn = functools.partial(
        pl.kernel,
        mesh=mesh,
        out_type=jax.ShapeDtypeStruct((_NW, _BINS), jnp.int32),
        scratch_types=[
            pltpu.VMEM((_SC_CHUNK,), jnp.int32),
            pltpu.VMEM((_BINS,), jnp.int32),
        ],
        compiler_params=pltpu.CompilerParams(needs_layout_passes=False),
    )(_sc_hist_body)
    return kfn(bits_flat)


def _tc2_body(bits_ref, hist_ref, dice_ref, out_ref):
    merged = jnp.sum(hist_ref[...], axis=0)
    r_idx = lax.broadcasted_iota(jnp.int32, (_BIN_ROWS, 128), 0)
    c_idx = lax.broadcasted_iota(jnp.int32, (_BIN_ROWS, 128), 1)
    b_idx = r_idx * 128 + c_idx

    def bin_step(_, carry):
        lo, hi = carry
        mid = lo + (hi - lo + 1) // 2
        ge = 1 - lax.shift_right_logical(b_idx - mid, 31)
        c_ge = jnp.sum(merged * ge)
        big = c_ge >= _K
        return jnp.where(big, mid, lo), jnp.where(big, hi, mid - 1)

    bstar, _ = lax.fori_loop(
        0, 16, bin_step, (jnp.int32(0), jnp.int32(_BINS - 1))
    )

    def step(_, carry):
        lo, hi = carry
        mid = lo + (hi - lo + 1) // 2
        parts = []
        for j in range(_SLICES):
            sl = bits_ref[pl.ds(j * _SROWS, _SROWS), :]
            parts.append(jnp.sum(lax.shift_right_logical(sl - mid, 31)))
        c_lt = sum(parts)
        big = c_lt <= _NMK
        lo = jnp.where(big, mid, lo)
        hi = jnp.where(big, hi, mid - 1)
        return lo, hi

    lo0 = bstar << 15
    lo, hi = lax.fori_loop(
        0, _REFINE_ITERS, step, (lo0, lo0 + jnp.int32(32767))
    )

    c_le_parts, s_parts, vk_parts = [], [], []
    for j in range(_SLICES):
        rows = pl.ds(j * _SROWS, _SROWS)
        bits = bits_ref[rows, :]
        b = pltpu.bitcast(bits, jnp.float32)
        le = lax.shift_right_logical(bits - (lo + 1), 31)
        c_le_parts.append(jnp.sum(le))
        s_parts.append(jnp.sum(b * (1 - le).astype(jnp.float32)))
        le_hi = lax.shift_right_logical(bits - (hi + 1), 31)
        vk_parts.append(jnp.max(b * le_hi.astype(jnp.float32)))
    c_gt = _N - sum(c_le_parts)
    s_gt = sum(s_parts)
    vk = jnp.max(jnp.stack(vk_parts))

    topk_mean = (s_gt + (_K - c_gt).astype(jnp.float32) * vk) / _K
    out_ref[...] = dice_ref[...] + topk_mean


def _tc2(bits, hist, dice):
    return pl.pallas_call(
        _tc2_body,
        out_shape=jax.ShapeDtypeStruct((1, 1), jnp.float32),
    )(bits, hist, dice)


def kernel(preds, gt_masks):
    p = preds.reshape(_ROWS, _COLS)
    t = gt_masks.reshape(_ROWS, _COLS)
    bits, dice = _tc1(p, t)
    hist = _sc_hist(bits.reshape(_N))
    out = _tc2(bits, hist.reshape(_NW, _BIN_ROWS, 128), dice)
    return out[0, 0]


# speedup vs baseline: 2.5209x; 2.5209x over previous
import jax
import jax.numpy as jnp
from jax.experimental import pallas as pl
from jax.experimental.pallas import tpu as pltpu

_N = 2097152
_K = 209715
_NMK = _N - _K
_ROWS = 2048
_COLS = 1024
_CHUNKS = 8
_CROWS = _ROWS // _CHUNKS
_HI_BITS = 0x42C80000
_HI_TOP = 0x42C8
_ITERS_A = 15
_ITERS_B = 8
_SLICES = 16
_SROWS = _ROWS // _SLICES


def _body(p_ref, t_ref, out_ref, bits_ref, top_ref, acc_ref):
    i = pl.program_id(0)

    @pl.when(i == 0)
    def _init():
        acc_ref[0] = 0.0
        acc_ref[1] = 0.0
        acc_ref[2] = 0.0

    sp, st, si = [], [], []
    for j in range(4):
        rows = pl.ds(j * (_CROWS // 4), _CROWS // 4)
        p = p_ref[rows, :]
        t = t_ref[rows, :]
        pt = p * t
        sp.append(jnp.sum(p))
        st.append(jnp.sum(t))
        si.append(jnp.sum(pt))
        q = (1.0 - p) - t + 2.0 * pt
        bce = jnp.maximum(jnp.minimum(-jnp.log(q), 100.0), 0.0)
        bits = pltpu.bitcast(bce, jnp.int32)
        grows = pl.ds(i * _CROWS + j * (_CROWS // 4), _CROWS // 4)
        bits_ref[grows, :] = bits
        top_ref[grows, :] = (bits >> 16).astype(jnp.int16)
    acc_ref[0] += sum(sp)
    acc_ref[1] += sum(st)
    acc_ref[2] += sum(si)

    @pl.when(i == _CHUNKS - 1)
    def _select():
        def step_a(_, carry):
            lo, hi = carry
            mid = lo + (hi - lo + 1) // 2
            mid16 = mid.astype(jnp.int16)
            parts = []
            for j in range(_SLICES):
                sl = top_ref[pl.ds(j * _SROWS, _SROWS), :]
                s = jnp.where(sl < mid16, jnp.int16(1), jnp.int16(0))
                r = _SROWS
                while r > 1:
                    r //= 2
                    s = s[:r] + s[r:]
                parts.append(s)
            total16 = parts[0]
            for prt in parts[1:]:
                total16 = total16 + prt
            c_lt = jnp.sum(total16.astype(jnp.int32))
            big = c_lt <= _NMK
            lo = jnp.where(big, mid, lo)
            hi = jnp.where(big, hi, mid - 1)
            return lo, hi

        top, _ = jax.lax.fori_loop(
            0, _ITERS_A, step_a, (jnp.int32(0), jnp.int32(_HI_TOP))
        )

        def step(_, carry):
            lo, hi = carry
            mid = lo + (hi - lo + 1) // 2
            parts = []
            for j in range(_SLICES):
                sl = bits_ref[pl.ds(j * _SROWS, _SROWS), :]
                parts.append(
                    jnp.sum(jax.lax.shift_right_logical(sl - mid, 31))
                )
            c_lt = sum(parts)
            big = c_lt <= _NMK
            lo = jnp.where(big, mid, lo)
            hi = jnp.where(big, hi, mid - 1)
            return lo, hi

        lo0 = top << 16
        lo, hi = jax.lax.fori_loop(
            0, _ITERS_B, step, (lo0, lo0 + jnp.int32(65535))
        )

        c_le_parts, s_parts, vk_parts = [], [], []
        for j in range(_SLICES):
            rows = pl.ds(j * _SROWS, _SROWS)
            bits = bits_ref[rows, :]
            b = pltpu.bitcast(bits, jnp.float32)
            le = jax.lax.shift_right_logical(bits - (lo + 1), 31)
            c_le_parts.append(jnp.sum(le))
            s_parts.append(jnp.sum(b * (1 - le).astype(jnp.float32)))
            le_hi = jax.lax.shift_right_logical(bits - (hi + 1), 31)
            vk_parts.append(jnp.max(b * le_hi.astype(jnp.float32)))
        c_gt = _N - sum(c_le_parts)
        s_gt = sum(s_parts)
        vk = jnp.max(jnp.stack(vk_parts))

        topk_mean = (s_gt + (_K - c_gt).astype(jnp.float32) * vk) / _K
        dice = 1.0 - (2.0 * acc_ref[2] + 1.0) / (acc_ref[0] + acc_ref[1] + 1.0)
        out_ref[...] = (dice + topk_mean).reshape(1, 1)


def kernel(preds, gt_masks):
    p = preds.reshape(_ROWS, _COLS)
    t = gt_masks.reshape(_ROWS, _COLS)
    out = pl.pallas_call(
        _body,
        grid=(_CHUNKS,),
        in_specs=[
            pl.BlockSpec((_CROWS, _COLS), lambda i: (i, 0)),
            pl.BlockSpec((_CROWS, _COLS), lambda i: (i, 0)),
        ],
        out_specs=pl.BlockSpec((1, 1), lambda i: (0, 0)),
        out_shape=jax.ShapeDtypeStruct((1, 1), jnp.float32),
        scratch_shapes=[
            pltpu.VMEM((_ROWS, _COLS), jnp.int32),
            pltpu.VMEM((_ROWS, _COLS), jnp.int16),
            pltpu.SMEM((4,), jnp.float32),
        ],
    )(p, t)
    return out[0, 0]
